# Initial kernel scaffold; baseline (speedup 1.0000x reference)
import jax
import jax.numpy as jnp
from jax.experimental import pallas as pl


def kernel(x, residual, w1, scale1, bias1, w2, scale2, bias2):
    raise NotImplementedError("write your pallas kernel here")


# speedup vs baseline: 1.3715x; 1.3715x over previous
import functools

import jax
import jax.numpy as jnp
from jax.experimental import pallas as pl
from jax.experimental.pallas import tpu as pltpu


def _bilinear_matrix(n_in, n_out):
    src = jnp.arange(n_out, dtype=jnp.float32) * (n_in - 1) / (n_out - 1)
    i0 = jnp.clip(jnp.floor(src).astype(jnp.int32), 0, n_in - 1)
    i1 = jnp.clip(i0 + 1, 0, n_in - 1)
    w1 = src - i0.astype(jnp.float32)
    w0 = 1.0 - w1
    rows = jnp.arange(n_out)
    A = jnp.zeros((n_out, n_in), jnp.float32)
    A = A.at[rows, i0].add(w0)
    A = A.at[rows, i1].add(w1)
    return A


def _fused_kernel(x_ref, res_ref, mt_ref, w1_ref, s1_ref, b1_ref,
                  w2_ref, s2_ref, b2_ref, o_ref, *, W1, S1):
    def shifted(v, off):
        return v if off == 0 else pltpu.roll(v, S1 - off, 1)

    def im2col(v):
        return jnp.concatenate(
            [shifted(v, kh * W1 + kw) for kh in range(3) for kw in range(3)],
            axis=0)

    up = jnp.dot(x_ref[0], mt_ref[...],
                 preferred_element_type=jnp.float32).astype(jnp.bfloat16)

    v = jnp.concatenate([res_ref[0], up], axis=0)
    acc1 = jnp.dot(w1_ref[...], im2col(v),
                   preferred_element_type=jnp.float32)
    y1 = jnp.maximum(acc1 * s1_ref[...] + b1_ref[...], 0.0).astype(jnp.bfloat16)

    acc2 = jnp.dot(w2_ref[...], im2col(y1),
                   preferred_element_type=jnp.float32)
    o_ref[0] = jnp.maximum(acc2 * s2_ref[...] + b2_ref[...],
                           0.0).astype(o_ref.dtype)


def kernel(x, residual, w1, scale1, bias1, w2, scale2, bias2):
    N, Cx, H, W = x.shape
    Cr, Hr, Wr = residual.shape[1], residual.shape[2], residual.shape[3]
    H1, W1 = 2 * H, 2 * W
    S1 = H1 * W1
    C1, C2 = w1.shape[1], w2.shape[1]

    ah = _bilinear_matrix(H, H1)
    aw = _bilinear_matrix(W, W1)
    mt = jnp.kron(ah, aw).T.astype(jnp.bfloat16)

    xf = x.reshape(N, Cx, H * W).astype(jnp.bfloat16)
    dy, dx = (Hr - H1) // 2, (Wr - W1) // 2
    res = residual[:, :, dy:Hr - dy, dx:Wr - dx].reshape(N, Cr, S1)
    res = res.astype(jnp.bfloat16)
    w1m = w1.transpose(1, 0, 2).reshape(C1, 9 * (Cr + Cx)).astype(jnp.bfloat16)
    w2m = w2.transpose(1, 0, 2).reshape(C2, 9 * C1).astype(jnp.bfloat16)
    s1 = scale1.reshape(C1, 1)
    b1 = bias1.reshape(C1, 1)
    s2 = scale2.reshape(C2, 1)
    b2 = bias2.reshape(C2, 1)

    fn = functools.partial(_fused_kernel, W1=W1, S1=S1)
    out = pl.pallas_call(
        fn,
        out_shape=jax.ShapeDtypeStruct((N, C2, S1), x.dtype),
        grid=(N,),
        in_specs=[
            pl.BlockSpec((1, Cx, H * W), lambda n: (n, 0, 0)),
            pl.BlockSpec((1, Cr, S1), lambda n: (n, 0, 0)),
            pl.BlockSpec((H * W, S1), lambda n: (0, 0)),
            pl.BlockSpec((C1, 9 * (Cr + Cx)), lambda n: (0, 0)),
            pl.BlockSpec((C1, 1), lambda n: (0, 0)),
            pl.BlockSpec((C1, 1), lambda n: (0, 0)),
            pl.BlockSpec((C2, 9 * C1), lambda n: (0, 0)),
            pl.BlockSpec((C2, 1), lambda n: (0, 0)),
            pl.BlockSpec((C2, 1), lambda n: (0, 0)),
        ],
        out_specs=pl.BlockSpec((1, C2, S1), lambda n: (n, 0, 0)),
        compiler_params=pltpu.CompilerParams(dimension_semantics=("parallel",)),
    )(xf, res, mt, w1m, s1, b1, w2m, s2, b2)
    return out.reshape(N, C2, H1, W1)[:, :, :H1 - 4, :W1 - 4]
